# baseline (speedup 1.0000x reference)
import functools

import jax
import jax.numpy as jnp
from jax import lax
from jax.experimental import pallas as pl
from jax.experimental.layout import Layout, with_layout_constraint
from jax.experimental.pallas import tpu as pltpu
from jax.experimental.pallas import tpu_sc as plsc

VOCAB = 100000
VOCAB_PAD = 102400
D_IN = 64
B = 4096
T = 200
NTOK = B * T


_COLS_BLK = 4096


def _scores_body(tabt_ref, w_ref, b_ref, out_ref):
    s = lax.dot_general(
        w_ref[...],
        tabt_ref[...],
        (((1,), (0,)), ((), ())),
        preferred_element_type=jnp.float32,
    )
    out_ref[...] = (s + b_ref[...])[0]


def _compute_scores(tab_t, wrow, b2d):
    return pl.pallas_call(
        _scores_body,
        grid=(VOCAB_PAD // _COLS_BLK,),
        in_specs=[
            pl.BlockSpec((D_IN, _COLS_BLK), lambda i: (0, i)),
            pl.BlockSpec((1, D_IN), lambda i: (0, 0)),
            pl.BlockSpec((1, 1), lambda i: (0, 0)),
        ],
        out_specs=pl.BlockSpec((_COLS_BLK,), lambda i: (i,)),
        out_shape=jax.ShapeDtypeStruct((VOCAB_PAD,), jnp.float32),
    )(tab_t, wrow, b2d)


_NC = 2
_NS = 16
_NW = _NC * _NS
_COLS_PER_W = B // _NW
_CHUNK_ROWS = 40
_LANES = 16


_N_CHUNKS = T // _CHUNK_ROWS


def _gather_body(
    scores_hbm, idst_hbm, outt_hbm,
    tab_v, idx_v0, idx_v1, out_v0, out_v1,
    sem_t, si0, si1, so0, so1,
):
    wid = lax.axis_index("s") * _NC + lax.axis_index("c")
    col0 = wid * _COLS_PER_W
    idx_bufs = (idx_v0, idx_v1)
    out_bufs = (out_v0, out_v1)
    isems = (si0, si1)
    osems = (so0, so1)

    def in_slice(c):
        return idst_hbm.at[
            pl.ds(c * _CHUNK_ROWS, _CHUNK_ROWS), pl.ds(col0, _COLS_PER_W)
        ]

    def out_slice(c):
        return outt_hbm.at[
            pl.ds(c * _CHUNK_ROWS, _CHUNK_ROWS), pl.ds(col0, _COLS_PER_W)
        ]

    tab_cp = pltpu.async_copy(scores_hbm, tab_v, sem_t)
    in_cps = [None] * _N_CHUNKS
    in_cps[0] = pltpu.async_copy(in_slice(0), idx_v0, si0)
    in_cps[1] = pltpu.async_copy(in_slice(1), idx_v1, si1)
    tab_cp.wait()
    out_cps = [None, None]
    for c in range(_N_CHUNKS):
        b = c & 1
        in_cps[c].wait()
        if out_cps[b] is not None:
            out_cps[b].wait()
        idx_v = idx_bufs[b]
        out_v = out_bufs[b]

        @plsc.parallel_loop(0, _CHUNK_ROWS, unroll=4)
        def _(r):
            for k in range(_COLS_PER_W // _LANES):
                ids = idx_v[r, pl.ds(_LANES * k, _LANES)]
                out_v[r, pl.ds(_LANES * k, _LANES)] = plsc.load_gather(
                    tab_v, [ids]
                )
        out_cps[b] = pltpu.async_copy(out_bufs[b], out_slice(c), osems[b])
        if c + 2 < _N_CHUNKS:
            in_cps[c + 2] = pltpu.async_copy(
                in_slice(c + 2), idx_bufs[b], isems[b]
            )
    for cp in out_cps:
        if cp is not None:
            cp.wait()


_gather_call = functools.partial(
    pl.kernel,
    out_type=jax.ShapeDtypeStruct((T, B), jnp.float32),
    mesh=plsc.VectorSubcoreMesh(core_axis_name="c", subcore_axis_name="s"),
    compiler_params=pltpu.CompilerParams(needs_layout_passes=False),
    scratch_types=[
        pltpu.VMEM((VOCAB_PAD,), jnp.float32),
        pltpu.VMEM((_CHUNK_ROWS, _COLS_PER_W), jnp.int32),
        pltpu.VMEM((_CHUNK_ROWS, _COLS_PER_W), jnp.int32),
        pltpu.VMEM((_CHUNK_ROWS, _COLS_PER_W), jnp.float32),
        pltpu.VMEM((_CHUNK_ROWS, _COLS_PER_W), jnp.float32),
        pltpu.SemaphoreType.DMA,
        pltpu.SemaphoreType.DMA,
        pltpu.SemaphoreType.DMA,
        pltpu.SemaphoreType.DMA,
        pltpu.SemaphoreType.DMA,
    ],
)


def kernel(input_ids, embed_table, proj_w, proj_b):
    tab_t = embed_table.T
    ids_t = input_ids.astype(jnp.int32).T
    b2d = proj_b.reshape(1, 1)
    scores = _compute_scores(tab_t, proj_w, b2d)
    out_t = _gather_call(_gather_body)(scores, ids_t)
    out = out_t.T[..., None]
    return with_layout_constraint(
        out, Layout(major_to_minor=(1, 2, 0), tiling=((1, 128),))
    )


# speedup vs baseline: 1.0021x; 1.0021x over previous
import functools

import jax
import jax.numpy as jnp
from jax import lax
from jax.experimental import pallas as pl
from jax.experimental.layout import Layout, with_layout_constraint
from jax.experimental.pallas import tpu as pltpu
from jax.experimental.pallas import tpu_sc as plsc

VOCAB = 100000
VOCAB_PAD = 102400
D_IN = 64
B = 4096
T = 200
NTOK = B * T


_COLS_BLK = 4096


def _scores_body(tabt_ref, w_ref, b_ref, out_ref):
    s = lax.dot_general(
        w_ref[...],
        tabt_ref[...],
        (((1,), (0,)), ((), ())),
        preferred_element_type=jnp.float32,
    )
    out_ref[...] = (s + b_ref[...])[0]


def _compute_scores(tab_t, wrow, b2d):
    return pl.pallas_call(
        _scores_body,
        grid=(VOCAB_PAD // _COLS_BLK,),
        in_specs=[
            pl.BlockSpec((D_IN, _COLS_BLK), lambda i: (0, i)),
            pl.BlockSpec((1, D_IN), lambda i: (0, 0)),
            pl.BlockSpec((1, 1), lambda i: (0, 0)),
        ],
        out_specs=pl.BlockSpec((_COLS_BLK,), lambda i: (i,)),
        out_shape=jax.ShapeDtypeStruct((VOCAB_PAD,), jnp.float32),
    )(tab_t, wrow, b2d)


_NC = 2
_NS = 16
_NW = _NC * _NS
_COLS_PER_W = B // _NW
_CHUNK_ROWS = 40
_LANES = 16


_N_CHUNKS = T // _CHUNK_ROWS


def _gather_body(
    scores_hbm, idst_hbm, outt_hbm,
    tab_v, idx_v0, idx_v1, out_v0, out_v1,
    sem_t, si0, si1, so0, so1,
):
    wid = lax.axis_index("s") * _NC + lax.axis_index("c")
    col0 = wid * _COLS_PER_W
    idx_bufs = (idx_v0, idx_v1)
    out_bufs = (out_v0, out_v1)
    isems = (si0, si1)
    osems = (so0, so1)

    def in_slice(c):
        return idst_hbm.at[
            pl.ds(c * _CHUNK_ROWS, _CHUNK_ROWS), pl.ds(col0, _COLS_PER_W)
        ]

    def out_slice(c):
        return outt_hbm.at[
            pl.ds(c * _CHUNK_ROWS, _CHUNK_ROWS), pl.ds(col0, _COLS_PER_W)
        ]

    tab_cp = pltpu.async_copy(scores_hbm, tab_v, sem_t)
    in_cps = [None] * _N_CHUNKS
    in_cps[0] = pltpu.async_copy(in_slice(0), idx_v0, si0)
    in_cps[1] = pltpu.async_copy(in_slice(1), idx_v1, si1)
    tab_cp.wait()
    out_cps = [None, None]
    for c in range(_N_CHUNKS):
        b = c & 1
        in_cps[c].wait()
        if out_cps[b] is not None:
            out_cps[b].wait()
        idx_v = idx_bufs[b]
        out_v = out_bufs[b]

        @plsc.parallel_loop(0, _CHUNK_ROWS, unroll=2)
        def _(r):
            for k in range(_COLS_PER_W // _LANES):
                ids = idx_v[r, pl.ds(_LANES * k, _LANES)]
                out_v[r, pl.ds(_LANES * k, _LANES)] = plsc.load_gather(
                    tab_v, [ids]
                )
        out_cps[b] = pltpu.async_copy(out_bufs[b], out_slice(c), osems[b])
        if c + 2 < _N_CHUNKS:
            in_cps[c + 2] = pltpu.async_copy(
                in_slice(c + 2), idx_bufs[b], isems[b]
            )
    for cp in out_cps:
        if cp is not None:
            cp.wait()


_gather_call = functools.partial(
    pl.kernel,
    out_type=jax.ShapeDtypeStruct((T, B), jnp.float32),
    mesh=plsc.VectorSubcoreMesh(core_axis_name="c", subcore_axis_name="s"),
    compiler_params=pltpu.CompilerParams(needs_layout_passes=False),
    scratch_types=[
        pltpu.VMEM((VOCAB_PAD,), jnp.float32),
        pltpu.VMEM((_CHUNK_ROWS, _COLS_PER_W), jnp.int32),
        pltpu.VMEM((_CHUNK_ROWS, _COLS_PER_W), jnp.int32),
        pltpu.VMEM((_CHUNK_ROWS, _COLS_PER_W), jnp.float32),
        pltpu.VMEM((_CHUNK_ROWS, _COLS_PER_W), jnp.float32),
        pltpu.SemaphoreType.DMA,
        pltpu.SemaphoreType.DMA,
        pltpu.SemaphoreType.DMA,
        pltpu.SemaphoreType.DMA,
        pltpu.SemaphoreType.DMA,
    ],
)


def kernel(input_ids, embed_table, proj_w, proj_b):
    tab_t = embed_table.T
    ids_t = input_ids.astype(jnp.int32).T
    b2d = proj_b.reshape(1, 1)
    scores = _compute_scores(tab_t, proj_w, b2d)
    out_t = _gather_call(_gather_body)(scores, ids_t)
    out = out_t.T[..., None]
    return with_layout_constraint(
        out, Layout(major_to_minor=(1, 2, 0), tiling=((1, 128),))
    )
